# Initial kernel scaffold; baseline (speedup 1.0000x reference)
import jax
import jax.numpy as jnp
from jax.experimental import pallas as pl


def kernel(x, edge_index, W1, b1, g1, bt1, W2, b2, g2, bt2, Wl, bl):
    raise NotImplementedError("write your pallas kernel here")


# speedup vs baseline: 14.9011x; 14.9011x over previous
import functools

import jax
import jax.numpy as jnp
from jax import lax
from jax.experimental import pallas as pl
from jax.experimental.pallas import tpu as pltpu
from jax.experimental.pallas import tpu_sc as plsc

N = 10000
E = 320000
NPAD = 10240
K = 128
NT = 16
NB_CONV = 160
EPT = K * NB_CONV
E_PAD = EPT * NT
ROWS_PT = NPAD // NT

_mesh = plsc.VectorSubcoreMesh(core_axis_name="c", subcore_axis_name="s")


def _deg_body(dst_hbm, ones_hbm, zvec_hbm, out_hbm, dbuf, ones_v, acc_sh, sem):
    del sem
    c = lax.axis_index("c")
    s = lax.axis_index("s")
    pltpu.sync_copy(ones_hbm, ones_v)
    pltpu.sync_copy(zvec_hbm, acc_sh.at[pl.ds(s * ROWS_PT, ROWS_PT)])
    plsc.subcore_barrier()

    nb = (E_PAD // K) // (2 * NT)
    e0 = (c * (E_PAD // 2)) + s * nb * K

    def body(b, _):
        pltpu.sync_copy(dst_hbm.at[pl.ds(e0 + b * K, K)], dbuf)
        pltpu.sync_copy(ones_v, acc_sh.at[dbuf], add=True)
        return 0

    lax.fori_loop(0, nb, body, 0)
    plsc.subcore_barrier()
    pltpu.sync_copy(acc_sh.at[pl.ds(s * ROWS_PT, ROWS_PT)],
                    out_hbm.at[pl.ds(c * NPAD + s * ROWS_PT, ROWS_PT)])


_deg_call = pl.kernel(
    _deg_body,
    out_type=jax.ShapeDtypeStruct((2 * NPAD,), jnp.float32),
    mesh=_mesh,
    scratch_types=[
        pltpu.VMEM((K,), jnp.int32),
        pltpu.VMEM((K,), jnp.float32),
        pltpu.VMEM_SHARED((NPAD,), jnp.float32),
        pltpu.SemaphoreType.DMA,
    ],
)


def _conv_body(nchunks, hs_hbm, srcs_hbm, dst_hbm, out_hbm,
               ibuf0, ibuf1, dbuf0, dbuf1, rbuf0, rbuf1, sem0, sem1, acc_sh):
    c = lax.axis_index("c")
    s = lax.axis_index("s")
    dst_e0 = s * EPT

    def load_fire(b, src0, ibuf, dbuf, rbuf, sem):
        pltpu.sync_copy(srcs_hbm.at[pl.ds(src0 + b * K, K)], ibuf)
        pltpu.sync_copy(dst_hbm.at[pl.ds(dst_e0 + b * K, K)], dbuf)
        pltpu.async_copy(hs_hbm.at[ibuf], rbuf, sem)

    def drain_scatter(dbuf, rbuf, sem):
        pltpu.make_async_copy(hs_hbm.at[pl.ds(0, K)], rbuf, sem).wait()
        pltpu.sync_copy(rbuf, acc_sh.at[dbuf], add=True)

    for r in range(nchunks // 2):
        ci = r * 2 + c
        src0 = ci * E_PAD + s * EPT

        pltpu.sync_copy(hs_hbm.at[pl.ds(ci * NPAD + s * ROWS_PT, ROWS_PT)],
                        acc_sh.at[pl.ds(s * ROWS_PT, ROWS_PT)])
        plsc.subcore_barrier()

        load_fire(0, src0, ibuf0, dbuf0, rbuf0, sem0)

        def pair(j, _):
            load_fire(2 * j + 1, src0, ibuf1, dbuf1, rbuf1, sem1)
            drain_scatter(dbuf0, rbuf0, sem0)

            @pl.when(j < NB_CONV // 2 - 1)
            def _():
                load_fire(2 * j + 2, src0, ibuf0, dbuf0, rbuf0, sem0)

            drain_scatter(dbuf1, rbuf1, sem1)
            return 0

        lax.fori_loop(0, NB_CONV // 2, pair, 0)
        plsc.subcore_barrier()
        pltpu.sync_copy(
            acc_sh.at[pl.ds(s * ROWS_PT, ROWS_PT)],
            out_hbm.at[pl.ds(ci * NPAD + s * ROWS_PT, ROWS_PT)])


def _make_conv_call(nchunks):
    return pl.kernel(
        functools.partial(_conv_body, nchunks),
        out_type=jax.ShapeDtypeStruct((nchunks * NPAD, 128), jnp.float32),
        mesh=_mesh,
        scratch_types=[
            pltpu.VMEM((K,), jnp.int32),
            pltpu.VMEM((K,), jnp.int32),
            pltpu.VMEM((K,), jnp.int32),
            pltpu.VMEM((K,), jnp.int32),
            pltpu.VMEM((K, 128), jnp.float32),
            pltpu.VMEM((K, 128), jnp.float32),
            pltpu.SemaphoreType.DMA,
            pltpu.SemaphoreType.DMA,
            pltpu.VMEM_SHARED((NPAD, 128), jnp.float32),
        ],
    )


_conv2_call = _make_conv_call(2)
_conv4_call = _make_conv_call(4)


def _dis_from_parts(degp_ref):
    deg = degp_ref[0, :] + degp_ref[1, :] + 1.0
    return lax.rsqrt(deg)


def _t1_body(degp_ref, x_ref, w_ref, out_ref):
    dis = _dis_from_parts(degp_ref)
    h = jnp.dot(x_ref[...], w_ref[...], preferred_element_type=jnp.float32)
    hs = h * dis[:, None]
    for ci in range(2):
        out_ref[pl.ds(ci * NPAD, N), :] = hs[:, ci * 128:(ci + 1) * 128]


def _mid_body(nc_in, nc_out, acc_ref, degp_ref, b_ref, g_ref, bt_ref,
              w_ref, out_ref):
    dis = _dis_from_parts(degp_ref)
    acc = jnp.concatenate(
        [acc_ref[pl.ds(ci * NPAD, N), :] for ci in range(nc_in)], axis=1)
    hconv = dis[:, None] * acc + b_ref[...]
    mean = jnp.mean(hconv, axis=0)
    xc = hconv - mean
    var = jnp.mean(xc * xc, axis=0)
    y = jnp.maximum(g_ref[...] * (xc * lax.rsqrt(var + 1e-5)) + bt_ref[...], 0.0)
    hs2 = jnp.dot(y, w_ref[...], preferred_element_type=jnp.float32)
    hs2 = hs2 * dis[:, None]
    for ci in range(nc_out):
        out_ref[pl.ds(ci * NPAD, N), :] = hs2[:, ci * 128:(ci + 1) * 128]


def _final_body(acc_ref, degp_ref, b_ref, g_ref, bt_ref, w_ref,
                bl_ref, out_ref):
    dis = _dis_from_parts(degp_ref)
    acc = jnp.concatenate(
        [acc_ref[pl.ds(ci * NPAD, N), :] for ci in range(4)], axis=1)
    hconv = dis[:, None] * acc + b_ref[...]
    mean = jnp.mean(hconv, axis=0)
    xc = hconv - mean
    var = jnp.mean(xc * xc, axis=0)
    y = jnp.maximum(g_ref[...] * (xc * lax.rsqrt(var + 1e-5)) + bt_ref[...], 0.0)
    out_ref[...] = (
        jnp.dot(y, w_ref[...], preferred_element_type=jnp.float32) + bl_ref[...])


def kernel(x, edge_index, W1, b1, g1, bt1, W2, b2, g2, bt2, Wl, bl):
    src = edge_index[0]
    dst = edge_index[1]
    pad = E_PAD - E

    pad_i = jnp.arange(pad, dtype=jnp.int32)
    dst_p = jnp.concatenate([dst, N + pad_i % (NPAD - N)])
    src_p = jnp.concatenate([src, pad_i % N])
    offs2 = (jnp.arange(2, dtype=jnp.int32) * NPAD)[:, None]
    offs4 = (jnp.arange(4, dtype=jnp.int32) * NPAD)[:, None]
    srcs2 = (src_p[None, :] + offs2).reshape(-1)
    srcs4 = (src_p[None, :] + offs4).reshape(-1)

    zvec = jnp.zeros((ROWS_PT,), jnp.float32)
    ones_blk = jnp.ones((K,), jnp.float32)

    degp = _deg_call(dst_p, ones_blk, zvec)
    degp2 = degp.reshape(2, NPAD)[:, :N]

    hs1 = pl.pallas_call(
        _t1_body,
        out_shape=jax.ShapeDtypeStruct((2 * NPAD, 128), jnp.float32),
    )(degp2, x, W1)

    acc1 = _conv2_call(hs1, srcs2, dst_p)

    hs2 = pl.pallas_call(
        functools.partial(_mid_body, 2, 4),
        out_shape=jax.ShapeDtypeStruct((4 * NPAD, 128), jnp.float32),
    )(acc1, degp2, b1, g1, bt1, W2)

    acc2 = _conv4_call(hs2, srcs4, dst_p)

    return pl.pallas_call(
        _final_body,
        out_shape=jax.ShapeDtypeStruct((N, 128), jnp.float32),
    )(acc2, degp2, b2, g2, bt2, Wl, bl)


# speedup vs baseline: 20.4647x; 1.3734x over previous
import functools

import jax
import jax.numpy as jnp
from jax import lax
from jax.experimental import pallas as pl
from jax.experimental.pallas import tpu as pltpu
from jax.experimental.pallas import tpu_sc as plsc

N = 10000
E = 320000
NPAD = 10240
K = 128
NT = 16
NB_CONV = 160
EPT = K * NB_CONV
E_PAD = EPT * NT
ROWS_PT = NPAD // NT

_mesh = plsc.VectorSubcoreMesh(core_axis_name="c", subcore_axis_name="s")


def _deg_body(dst_hbm, ones_hbm, zvec_hbm, out_hbm, dbig, ones_v, acc_sh, sem):
    del sem
    c = lax.axis_index("c")
    s = lax.axis_index("s")
    nb = (E_PAD // K) // (2 * NT)
    pltpu.sync_copy(ones_hbm, ones_v)
    pltpu.sync_copy(dst_hbm.at[pl.ds(c * (NT * nb) + s * nb, nb)], dbig)
    pltpu.sync_copy(zvec_hbm, acc_sh.at[pl.ds(s * ROWS_PT, ROWS_PT)])
    plsc.subcore_barrier()

    def body(b, _):
        pltpu.sync_copy(ones_v, acc_sh.at[dbig.at[b]], add=True)
        return 0

    lax.fori_loop(0, nb, body, 0)
    plsc.subcore_barrier()
    pltpu.sync_copy(acc_sh.at[pl.ds(s * ROWS_PT, ROWS_PT)],
                    out_hbm.at[pl.ds(c * NPAD + s * ROWS_PT, ROWS_PT)])


_deg_call = pl.kernel(
    _deg_body,
    out_type=jax.ShapeDtypeStruct((2 * NPAD,), jnp.float32),
    mesh=_mesh,
    scratch_types=[
        pltpu.VMEM((80, K), jnp.int32),
        pltpu.VMEM((K,), jnp.float32),
        pltpu.VMEM_SHARED((NPAD,), jnp.float32),
        pltpu.SemaphoreType.DMA,
    ],
)


NRING = 2
NBB = 40


def _conv_body(nchunks, hs_hbm, srcs_hbm, dst_hbm, out_hbm,
               sbig, dbig, rb0, rb1, sm0, sm1, acc_sh):
    c = lax.axis_index("c")
    s = lax.axis_index("s")
    rbufs = (rb0, rb1)
    sems = (sm0, sm1)

    for r in range(nchunks // 2):
        ci = r * 2 + c
        pltpu.sync_copy(hs_hbm.at[pl.ds(ci * NPAD + s * ROWS_PT, ROWS_PT)],
                        acc_sh.at[pl.ds(s * ROWS_PT, ROWS_PT)])
        plsc.subcore_barrier()

        for h in range(NB_CONV // NBB):
            pltpu.sync_copy(
                srcs_hbm.at[pl.ds(
                    ci * (NT * NB_CONV) + s * NB_CONV + h * NBB, NBB)],
                sbig)
            pltpu.sync_copy(
                dst_hbm.at[pl.ds(s * NB_CONV + h * NBB, NBB)], dbig)

            for p in range(NRING):
                pltpu.async_copy(hs_hbm.at[sbig.at[p]], rbufs[p], sems[p])

            def ring(j, _):
                for p in range(NRING):
                    b = NRING * j + p
                    pltpu.make_async_copy(
                        hs_hbm.at[pl.ds(0, K)], rbufs[p], sems[p]).wait()
                    pltpu.sync_copy(rbufs[p], acc_sh.at[dbig.at[b]], add=True)

                    @pl.when(b + NRING < NBB)
                    def _():
                        pltpu.async_copy(
                            hs_hbm.at[sbig.at[b + NRING]], rbufs[p], sems[p])
                return 0

            lax.fori_loop(0, NBB // NRING, ring, 0)

        plsc.subcore_barrier()
        pltpu.sync_copy(
            acc_sh.at[pl.ds(s * ROWS_PT, ROWS_PT)],
            out_hbm.at[pl.ds(ci * NPAD + s * ROWS_PT, ROWS_PT)])


def _make_conv_call(nchunks):
    return pl.kernel(
        functools.partial(_conv_body, nchunks),
        out_type=jax.ShapeDtypeStruct((nchunks * NPAD, 128), jnp.float32),
        mesh=_mesh,
        scratch_types=[
            pltpu.VMEM((NBB, K), jnp.int32),
            pltpu.VMEM((NBB, K), jnp.int32),
            pltpu.VMEM((K, 128), jnp.float32),
            pltpu.VMEM((K, 128), jnp.float32),
            pltpu.SemaphoreType.DMA,
            pltpu.SemaphoreType.DMA,
            pltpu.VMEM_SHARED((NPAD, 128), jnp.float32),
        ],
    )


_conv2_call = _make_conv_call(2)
_conv4_call = _make_conv_call(4)


def _dis_from_parts(degp_ref):
    deg = degp_ref[0, :] + degp_ref[1, :] + 1.0
    return lax.rsqrt(deg)


def _t1_body(degp_ref, x_ref, w_ref, out_ref):
    dis = _dis_from_parts(degp_ref)
    h = jnp.dot(x_ref[...], w_ref[...], preferred_element_type=jnp.float32)
    hs = h * dis[:, None]
    for ci in range(2):
        out_ref[pl.ds(ci * NPAD, N), :] = hs[:, ci * 128:(ci + 1) * 128]


def _mid_body(nc_in, nc_out, acc_ref, degp_ref, b_ref, g_ref, bt_ref,
              w_ref, out_ref):
    dis = _dis_from_parts(degp_ref)
    acc = jnp.concatenate(
        [acc_ref[pl.ds(ci * NPAD, N), :] for ci in range(nc_in)], axis=1)
    hconv = dis[:, None] * acc + b_ref[...]
    mean = jnp.mean(hconv, axis=0)
    xc = hconv - mean
    var = jnp.mean(xc * xc, axis=0)
    y = jnp.maximum(g_ref[...] * (xc * lax.rsqrt(var + 1e-5)) + bt_ref[...], 0.0)
    hs2 = jnp.dot(y, w_ref[...], preferred_element_type=jnp.float32)
    hs2 = hs2 * dis[:, None]
    for ci in range(nc_out):
        out_ref[pl.ds(ci * NPAD, N), :] = hs2[:, ci * 128:(ci + 1) * 128]


def _final_body(acc_ref, degp_ref, b_ref, g_ref, bt_ref, w_ref,
                bl_ref, out_ref):
    dis = _dis_from_parts(degp_ref)
    acc = jnp.concatenate(
        [acc_ref[pl.ds(ci * NPAD, N), :] for ci in range(4)], axis=1)
    hconv = dis[:, None] * acc + b_ref[...]
    mean = jnp.mean(hconv, axis=0)
    xc = hconv - mean
    var = jnp.mean(xc * xc, axis=0)
    y = jnp.maximum(g_ref[...] * (xc * lax.rsqrt(var + 1e-5)) + bt_ref[...], 0.0)
    out_ref[...] = (
        jnp.dot(y, w_ref[...], preferred_element_type=jnp.float32) + bl_ref[...])


def kernel(x, edge_index, W1, b1, g1, bt1, W2, b2, g2, bt2, Wl, bl):
    src = edge_index[0]
    dst = edge_index[1]
    pad = E_PAD - E

    pad_i = jnp.arange(pad, dtype=jnp.int32)
    dst_p = jnp.concatenate([dst, N + pad_i % (NPAD - N)]).reshape(E_PAD // K, K)
    src_p = jnp.concatenate([src, pad_i % N])
    offs2 = (jnp.arange(2, dtype=jnp.int32) * NPAD)[:, None]
    offs4 = (jnp.arange(4, dtype=jnp.int32) * NPAD)[:, None]
    srcs2 = (src_p[None, :] + offs2).reshape(2 * E_PAD // K, K)
    srcs4 = (src_p[None, :] + offs4).reshape(4 * E_PAD // K, K)

    zvec = jnp.zeros((ROWS_PT,), jnp.float32)
    ones_blk = jnp.ones((K,), jnp.float32)

    degp = _deg_call(dst_p, ones_blk, zvec)
    degp2 = degp.reshape(2, NPAD)[:, :N]

    hs1 = pl.pallas_call(
        _t1_body,
        out_shape=jax.ShapeDtypeStruct((2 * NPAD, 128), jnp.float32),
    )(degp2, x, W1)

    acc1 = _conv2_call(hs1, srcs2, dst_p)

    hs2 = pl.pallas_call(
        functools.partial(_mid_body, 2, 4),
        out_shape=jax.ShapeDtypeStruct((4 * NPAD, 128), jnp.float32),
    )(acc1, degp2, b1, g1, bt1, W2)

    acc2 = _conv4_call(hs2, srcs4, dst_p)

    return pl.pallas_call(
        _final_body,
        out_shape=jax.ShapeDtypeStruct((N, 128), jnp.float32),
    )(acc2, degp2, b2, g2, bt2, Wl, bl)


# speedup vs baseline: 20.4969x; 1.0016x over previous
import functools

import jax
import jax.numpy as jnp
from jax import lax
from jax.experimental import pallas as pl
from jax.experimental.pallas import tpu as pltpu
from jax.experimental.pallas import tpu_sc as plsc

N = 10000
E = 320000
NPAD = 10240
K = 128
NT = 16
NB_CONV = 160
EPT = K * NB_CONV
E_PAD = EPT * NT
ROWS_PT = NPAD // NT

_mesh = plsc.VectorSubcoreMesh(core_axis_name="c", subcore_axis_name="s")


def _deg_body(dst_hbm, ones_hbm, zvec_hbm, out_hbm, dbig, ones_v, acc_sh, sem):
    del sem
    c = lax.axis_index("c")
    s = lax.axis_index("s")
    nb = (E_PAD // K) // (2 * NT)
    pltpu.sync_copy(ones_hbm, ones_v)
    pltpu.sync_copy(dst_hbm.at[pl.ds(c * (NT * nb) + s * nb, nb)], dbig)
    pltpu.sync_copy(zvec_hbm, acc_sh.at[pl.ds(s * ROWS_PT, ROWS_PT)])
    plsc.subcore_barrier()

    def body(b, _):
        pltpu.sync_copy(ones_v, acc_sh.at[dbig.at[b]], add=True)
        return 0

    lax.fori_loop(0, nb, body, 0)
    plsc.subcore_barrier()
    pltpu.sync_copy(acc_sh.at[pl.ds(s * ROWS_PT, ROWS_PT)],
                    out_hbm.at[pl.ds(c * NPAD + s * ROWS_PT, ROWS_PT)])


_deg_call = pl.kernel(
    _deg_body,
    out_type=jax.ShapeDtypeStruct((2 * NPAD,), jnp.float32),
    mesh=_mesh,
    scratch_types=[
        pltpu.VMEM(((E_PAD // K) // (2 * NT), K), jnp.int32),
        pltpu.VMEM((K,), jnp.float32),
        pltpu.VMEM_SHARED((NPAD,), jnp.float32),
        pltpu.SemaphoreType.DMA,
    ],
)


NRING = 2
NBB = 40


def _conv_body(nchunks, hs_hbm, srcs_hbm, dst_hbm, out_hbm,
               sbig, dbig, rb0, rb1, sm0, sm1, acc_sh):
    c = lax.axis_index("c")
    s = lax.axis_index("s")
    rbufs = (rb0, rb1)
    sems = (sm0, sm1)

    for r in range(nchunks // 2):
        ci = r * 2 + c
        pltpu.sync_copy(hs_hbm.at[pl.ds(ci * NPAD + s * ROWS_PT, ROWS_PT)],
                        acc_sh.at[pl.ds(s * ROWS_PT, ROWS_PT)])
        plsc.subcore_barrier()

        for h in range(NB_CONV // NBB):
            pltpu.sync_copy(
                srcs_hbm.at[pl.ds(
                    ci * (NT * NB_CONV) + s * NB_CONV + h * NBB, NBB)],
                sbig)
            pltpu.sync_copy(
                dst_hbm.at[pl.ds(s * NB_CONV + h * NBB, NBB)], dbig)

            for p in range(NRING):
                pltpu.async_copy(hs_hbm.at[sbig.at[p]], rbufs[p], sems[p])

            def ring(j, _):
                for p in range(NRING):
                    b = NRING * j + p
                    pltpu.make_async_copy(
                        hs_hbm.at[pl.ds(0, K)], rbufs[p], sems[p]).wait()
                    pltpu.sync_copy(rbufs[p], acc_sh.at[dbig.at[b]], add=True)

                    @pl.when(b + NRING < NBB)
                    def _():
                        pltpu.async_copy(
                            hs_hbm.at[sbig.at[b + NRING]], rbufs[p], sems[p])
                return 0

            lax.fori_loop(0, NBB // NRING, ring, 0)

        plsc.subcore_barrier()
        pltpu.sync_copy(
            acc_sh.at[pl.ds(s * ROWS_PT, ROWS_PT)],
            out_hbm.at[pl.ds(ci * NPAD + s * ROWS_PT, ROWS_PT)])


def _make_conv_call(nchunks):
    return pl.kernel(
        functools.partial(_conv_body, nchunks),
        out_type=jax.ShapeDtypeStruct((nchunks * NPAD, 128), jnp.float32),
        mesh=_mesh,
        scratch_types=[
            pltpu.VMEM((NBB, K), jnp.int32),
            pltpu.VMEM((NBB, K), jnp.int32),
            pltpu.VMEM((K, 128), jnp.float32),
            pltpu.VMEM((K, 128), jnp.float32),
            pltpu.SemaphoreType.DMA,
            pltpu.SemaphoreType.DMA,
            pltpu.VMEM_SHARED((NPAD, 128), jnp.float32),
        ],
    )


_conv2_call = _make_conv_call(2)
_conv4_call = _make_conv_call(4)


def _dis_from_parts(degp_ref):
    deg = degp_ref[0, :] + degp_ref[1, :] + 1.0
    return lax.rsqrt(deg)


def _t1_body(degp_ref, x_ref, w_ref, out_ref):
    dis = _dis_from_parts(degp_ref)
    h = jnp.dot(x_ref[...], w_ref[...], preferred_element_type=jnp.float32)
    hs = h * dis[:, None]
    for ci in range(2):
        out_ref[pl.ds(ci * NPAD, N), :] = hs[:, ci * 128:(ci + 1) * 128]


def _mid_body(nc_in, nc_out, acc_ref, degp_ref, b_ref, g_ref, bt_ref,
              w_ref, out_ref):
    dis = _dis_from_parts(degp_ref)
    acc = jnp.concatenate(
        [acc_ref[pl.ds(ci * NPAD, N), :] for ci in range(nc_in)], axis=1)
    hconv = dis[:, None] * acc + b_ref[...]
    mean = jnp.mean(hconv, axis=0)
    xc = hconv - mean
    var = jnp.mean(xc * xc, axis=0)
    y = jnp.maximum(g_ref[...] * (xc * lax.rsqrt(var + 1e-5)) + bt_ref[...], 0.0)
    hs2 = jnp.dot(y, w_ref[...], preferred_element_type=jnp.float32)
    hs2 = hs2 * dis[:, None]
    for ci in range(nc_out):
        out_ref[pl.ds(ci * NPAD, N), :] = hs2[:, ci * 128:(ci + 1) * 128]


def _final_body(acc_ref, degp_ref, b_ref, g_ref, bt_ref, w_ref,
                bl_ref, out_ref):
    dis = _dis_from_parts(degp_ref)
    acc = jnp.concatenate(
        [acc_ref[pl.ds(ci * NPAD, N), :] for ci in range(4)], axis=1)
    hconv = dis[:, None] * acc + b_ref[...]
    mean = jnp.mean(hconv, axis=0)
    xc = hconv - mean
    var = jnp.mean(xc * xc, axis=0)
    y = jnp.maximum(g_ref[...] * (xc * lax.rsqrt(var + 1e-5)) + bt_ref[...], 0.0)
    out_ref[...] = (
        jnp.dot(y, w_ref[...], preferred_element_type=jnp.float32) + bl_ref[...])


def kernel(x, edge_index, W1, b1, g1, bt1, W2, b2, g2, bt2, Wl, bl):
    src = edge_index[0]
    dst = edge_index[1]
    pad = E_PAD - E

    pad_i = jnp.arange(pad, dtype=jnp.int32)
    dst_p = jnp.concatenate([dst, N + pad_i % (NPAD - N)]).reshape(E_PAD // K, K)
    src_p = jnp.concatenate([src, pad_i % N])
    offs2 = (jnp.arange(2, dtype=jnp.int32) * NPAD)[:, None]
    offs4 = (jnp.arange(4, dtype=jnp.int32) * NPAD)[:, None]
    srcs2 = (src_p[None, :] + offs2).reshape(2 * E_PAD // K, K)
    srcs4 = (src_p[None, :] + offs4).reshape(4 * E_PAD // K, K)

    zvec = jnp.zeros((ROWS_PT,), jnp.float32)
    ones_blk = jnp.ones((K,), jnp.float32)

    degp = _deg_call(dst_p, ones_blk, zvec)
    degp2 = degp.reshape(2, NPAD)[:, :N]

    hs1 = pl.pallas_call(
        _t1_body,
        out_shape=jax.ShapeDtypeStruct((2 * NPAD, 128), jnp.float32),
    )(degp2, x, W1)

    acc1 = _conv2_call(hs1, srcs2, dst_p)

    hs2 = pl.pallas_call(
        functools.partial(_mid_body, 2, 4),
        out_shape=jax.ShapeDtypeStruct((4 * NPAD, 128), jnp.float32),
    )(acc1, degp2, b1, g1, bt1, W2)

    acc2 = _conv4_call(hs2, srcs4, dst_p)

    return pl.pallas_call(
        _final_body,
        out_shape=jax.ShapeDtypeStruct((N, 128), jnp.float32),
    )(acc2, degp2, b2, g2, bt2, Wl, bl)
